# baseline (speedup 1.0000x reference)
import functools

import jax
import jax.numpy as jnp
from jax import lax
from jax.experimental import pallas as pl
from jax.experimental.pallas import tpu as pltpu
from jax.experimental.pallas import tpu_sc as plsc

_N_PAD = 10240
_NEG = 0.2
_CH = 1024


def _mm0_body(x_ref, w_ref, as_ref, ad_ref, o_ref, t_ref):
    h = jnp.dot(x_ref[...], w_ref[...], preferred_element_type=jnp.float32)
    als = jnp.dot(h, as_ref[...], preferred_element_type=jnp.float32)
    ald = jnp.dot(h, ad_ref[...], preferred_element_type=jnp.float32)
    bn, m = h.shape
    d_row = o_ref.shape[1]
    z = jnp.zeros((bn, 8 - als.shape[1]), jnp.float32)
    tail = jnp.concatenate([als, z, ald, z], axis=1)
    t_ref[...] = jnp.concatenate([tail, jnp.zeros((bn, 16), jnp.float32)],
                                 axis=1)
    parts = [h, tail]
    if d_row > m + 16:
        parts.append(jnp.zeros((bn, d_row - m - 16), jnp.float32))
    o_ref[...] = jnp.concatenate(parts, axis=1)


def _mm0(x, w, a_s, a_d, bn=512):
    n, k = x.shape
    m = w.shape[1]
    heads = a_s.shape[1]
    d_row = ((m + 16 + 127) // 128) * 128
    return pl.pallas_call(
        _mm0_body,
        grid=(n // bn,),
        in_specs=[pl.BlockSpec((bn, k), lambda i: (i, 0)),
                  pl.BlockSpec((k, m), lambda i: (0, 0)),
                  pl.BlockSpec((m, heads), lambda i: (0, 0)),
                  pl.BlockSpec((m, heads), lambda i: (0, 0))],
        out_specs=[pl.BlockSpec((bn, d_row), lambda i: (i, 0)),
                   pl.BlockSpec((bn, 32), lambda i: (i, 0))],
        out_shape=[jax.ShapeDtypeStruct((n, d_row), jnp.float32),
                   jax.ShapeDtypeStruct((n, 32), jnp.float32)],
    )(x, w, a_s, a_d)


def _mid_body(x_ref, b_ref, g_ref, be_ref, w_ref, as_ref, ad_ref,
              o_ref, t_ref):
    t = x_ref[...] + b_ref[...]
    mu = jnp.mean(t, axis=-1, keepdims=True)
    var = jnp.mean((t - mu) * (t - mu), axis=-1, keepdims=True)
    t = (t - mu) * lax.rsqrt(var + 1e-5) * g_ref[...] + be_ref[...]
    t = jnp.where(t > 0, t, jnp.exp(jnp.minimum(t, 0.0)) - 1.0)
    h = jnp.dot(t, w_ref[...], preferred_element_type=jnp.float32)
    als = jnp.dot(h, as_ref[...], preferred_element_type=jnp.float32)
    ald = jnp.dot(h, ad_ref[...], preferred_element_type=jnp.float32)
    bn, m = h.shape
    d_row = o_ref.shape[1]
    z = jnp.zeros((bn, 8 - als.shape[1]), jnp.float32)
    tail = jnp.concatenate([als, z, ald, z], axis=1)
    t_ref[...] = jnp.concatenate([tail, jnp.zeros((bn, 16), jnp.float32)],
                                 axis=1)
    parts = [h, tail]
    if d_row > m + 16:
        parts.append(jnp.zeros((bn, d_row - m - 16), jnp.float32))
    o_ref[...] = jnp.concatenate(parts, axis=1)


def _mid(x, b, g, be, w, a_s, a_d, bn=512):
    n, k = x.shape
    m = w.shape[1]
    heads = a_s.shape[1]
    d_row = ((m + 16 + 127) // 128) * 128
    vec = pl.BlockSpec((1, k), lambda i: (0, 0))
    return pl.pallas_call(
        _mid_body,
        grid=(n // bn,),
        in_specs=[pl.BlockSpec((bn, k), lambda i: (i, 0)), vec, vec, vec,
                  pl.BlockSpec((k, m), lambda i: (0, 0)),
                  pl.BlockSpec((m, heads), lambda i: (0, 0)),
                  pl.BlockSpec((m, heads), lambda i: (0, 0))],
        out_specs=[pl.BlockSpec((bn, d_row), lambda i: (i, 0)),
                   pl.BlockSpec((bn, 32), lambda i: (i, 0))],
        out_shape=[jax.ShapeDtypeStruct((n, d_row), jnp.float32),
                   jax.ShapeDtypeStruct((n, 32), jnp.float32)],
    )(x, b, g, be, w, a_s, a_d)


def _final_body(x_ref, b_ref, o_ref):
    s = x_ref[:, 0:40]
    for j in range(1, 6):
        s = s + x_ref[:, 40 * j:40 * j + 40]
    t = s * (1.0 / 6.0) + b_ref[...]
    m = jnp.max(t, axis=-1, keepdims=True)
    u = t - m
    o_ref[...] = u - jnp.log(jnp.sum(jnp.exp(u), axis=-1, keepdims=True))


def _final(x, b, bn=1024):
    n = x.shape[0]
    return pl.pallas_call(
        _final_body,
        grid=(n // bn,),
        in_specs=[pl.BlockSpec((bn, 256), lambda i: (i, 0)),
                  pl.BlockSpec((1, 40), lambda i: (0, 0))],
        out_specs=pl.BlockSpec((bn, 40), lambda i: (i, 0)),
        out_shape=jax.ShapeDtypeStruct((n, 40), jnp.float32),
    )(x, b)


def _sc_layer(hext, altail, pk, *, heads, d_feat, rpr, n_ranges, e_pad):
    d_row = hext.shape[1]
    hs = 4 if heads <= 4 else 8
    nv = d_feat // 16
    uniform = (d_feat // heads) % 16 == 0
    vph = (d_feat // heads) // 16 if uniform else 0
    rpt = n_ranges // 32
    nch = e_pad // _CH
    qcap = _CH + 64
    qdump = _CH + 48

    mesh = plsc.VectorSubcoreMesh(core_axis_name="c", subcore_axis_name="s",
                                  num_cores=2, num_subcores=16)

    @functools.partial(
        pl.kernel,
        out_type=jax.ShapeDtypeStruct((_N_PAD, d_feat), jnp.float32),
        mesh=mesh,
        compiler_params=pltpu.CompilerParams(needs_layout_passes=False),
        scratch_types=[
            pltpu.VMEM((_CH,), jnp.int32),
            pltpu.VMEM((_CH,), jnp.int32),
            pltpu.VMEM((qcap,), jnp.int32),
            pltpu.VMEM((16,), jnp.int32),
            pltpu.VMEM((16,), jnp.int32),
            pltpu.VMEM((32,), jnp.int32),
            pltpu.VMEM((32,), jnp.int32),
            pltpu.VMEM((16, d_row), jnp.float32),
            pltpu.VMEM((16, d_row), jnp.float32),
            pltpu.VMEM((rpr, 32), jnp.float32),
            pltpu.VMEM((rpr, d_feat), jnp.float32),
            pltpu.VMEM((rpr * hs + 32,), jnp.float32),
            pltpu.VMEM((32,), jnp.float32),
            pltpu.VMEM((32,), jnp.float32),
            pltpu.VMEM((d_feat + 32,), jnp.float32),
            pltpu.VMEM((64,), jnp.int32),
            pltpu.SemaphoreType.DMA,
            pltpu.SemaphoreType.DMA,
            pltpu.SemaphoreType.DMA,
            pltpu.SemaphoreType.DMA,
        ],
    )
    def k(hext_hbm, altail_hbm, pk_hbm, out_hbm, pka, pkb, cpk, csrca,
          csrcb, cdla, cdlb, rowsa, rowsb, ald2d, outb, denom, pe,
          dinv, wbuf, ps, sema, semb, semg, semh):
        wid = lax.axis_index("s") * 2 + lax.axis_index("c")
        iota = lax.iota(jnp.int32, 16)
        zf = jnp.zeros((16,), jnp.float32)
        zi = jnp.zeros((16,), jnp.int32)

        def _zq(i, _):
            cpk[pl.ds(i * 16, 16)] = zi
            return 0
        lax.fori_loop(0, qcap // 16, _zq, 0)
        for i in range(3):
            ps[pl.ds(i * 16, 16)] = zi

        def range_body(kk, _):
            r = wid * rpt + kk
            d0 = r * rpr

            def _zo(i, _):
                for v in range(nv):
                    outb[i, pl.ds(v * 16, 16)] = zf
                return 0
            lax.fori_loop(0, rpr, _zo, 0)

            def _zd(i, _):
                denom[pl.ds(i * 16, 16)] = zf
                return 0
            lax.fori_loop(0, (rpr * hs + 32) // 16, _zd, 0)

            pltpu.sync_copy(altail_hbm.at[pl.ds(d0, rpr)], ald2d)

            lo = d0 << 16

            def unpack_fire(b, cs, cd, rw, sm):
                cpkv = cpk[pl.ds(b * 16, 16)]
                cs[...] = lax.bitwise_and(cpkv, 0xFFFF)
                cd[pl.ds(0, 16)] = lax.shift_right_logical(cpkv, 16) - d0
                pltpu.async_copy(hext_hbm.at[cs], rw, sm)

            def edge_work(cnt, cd, rw):
                def edge(j, _):
                    iot = lax.iota(jnp.int32, 16)
                    hmk = jnp.where(iot < heads, jnp.int32(1), jnp.int32(0))
                    zj = j * 0
                    dl = cd[pl.ds(j, 16)][0]
                    tail = rw[j, pl.ds(d_feat, 16)]
                    aldv = ald2d[dl, pl.ds(8, 16)]
                    e = tail + aldv
                    e = jnp.maximum(e, 0.0) + _NEG * jnp.minimum(e, 0.0)
                    p = jnp.exp(jnp.minimum(e, 60.0))
                    didx = rpr * hs + iot + hmk * ((dl - rpr) * hs)
                    plsc.addupdate_scatter(denom, [didx], p)
                    pe[pl.ds(0, 16)] = p
                    if uniform:
                        for hv in range(heads):
                            ah = pe[pl.ds(zj + hv, 16)][0]
                            for v in range(hv * vph, (hv + 1) * vph):
                                plsc.addupdate(
                                    outb.at[dl, pl.ds(v * 16, 16)],
                                    rw[j, pl.ds(v * 16, 16)] * ah)
                    else:
                        cph = d_feat // heads
                        nst = (cph + 15) // 16
                        for hv in range(heads):
                            ah = pe[pl.ds(zj + hv, 16)][0]
                            av = jnp.full((16,), ah, jnp.float32)
                            for t in range(nst):
                                wbuf[pl.ds(zj + hv * cph + t * 16, 16)] = av
                        for v in range(nv):
                            plsc.addupdate(
                                outb.at[dl, pl.ds(v * 16, 16)],
                                rw[j, pl.ds(v * 16, 16)]
                                * wbuf[pl.ds(v * 16, 16)])
                    return 0
                lax.fori_loop(0, cnt, edge, 0)

            def consume(cnt, cs, cd, rw, sm):
                pltpu.make_async_copy(hext_hbm.at[cs], rw, sm).wait()
                edge_work(cnt, cd, rw)

            pltpu.async_copy(pk_hbm.at[pl.ds(0, _CH)], pka, sema)

            def two_chunks(i, nc):
                for bsel in range(2):
                    buf = pka if bsel == 0 else pkb
                    sem = sema if bsel == 0 else semb
                    obuf = pkb if bsel == 0 else pka
                    osem = semb if bsel == 0 else sema
                    c = i * 2 + bsel
                    pltpu.make_async_copy(
                        pk_hbm.at[pl.ds(c * _CH, _CH)], buf, sem).wait()

                    @pl.when(c + 1 < nch)
                    def _():
                        pltpu.async_copy(
                            pk_hbm.at[pl.ds((c + 1) * _CH, _CH)], obuf, osem)

                    def compact(g, nc2):
                        iot = lax.iota(jnp.int32, 16)
                        v = buf[pl.ds(g * 16, 16)]
                        u = (v - lo).astype(jnp.uint32)
                        m = u < jnp.uint32(rpr << 16)
                        mi = jnp.where(m, jnp.int32(1), jnp.int32(0))
                        acc = mi
                        z = nc2 * 0
                        for kk in (1, 2, 4, 8):
                            ps[pl.ds(8, 16)] = acc
                            acc = acc + ps[pl.ds(z + (8 - kk), 16)]
                        ps[pl.ds(32, 16)] = acc
                        base = qdump + iot
                        idx = base + mi * (nc2 + acc - 1 - base)
                        plsc.store_scatter(cpk, [idx], v)
                        return nc2 + ps[pl.ds(z + 47, 16)][0]
                    nc = lax.fori_loop(0, _CH // 16, compact, nc)

                    nbat = lax.div(nc, jnp.int32(16))

                    @pl.when(nbat > 0)
                    def _():
                        unpack_fire(0, csrca, cdla, rowsa, semg)

                    def bat2(i, _):
                        for s in range(2):
                            if s == 0:
                                cs, cd, rw, sm = csrca, cdla, rowsa, semg
                                ns, nd, nr, nm = csrcb, cdlb, rowsb, semh
                            else:
                                cs, cd, rw, sm = csrcb, cdlb, rowsb, semh
                                ns, nd, nr, nm = csrca, cdla, rowsa, semg
                            bb = i * 2 + s

                            @pl.when(bb < nbat)
                            def _():
                                @pl.when(bb + 1 < nbat)
                                def _():
                                    unpack_fire(bb + 1, ns, nd, nr, nm)
                                consume(16, cs, cd, rw, sm)
                        return 0
                    lax.fori_loop(0, lax.div(nbat + 1, jnp.int32(2)),
                                  bat2, 0)

                    lv = cpk[pl.ds(nbat * 16, 16)]
                    cpk[pl.ds(0, 16)] = lv
                    nc = nc - nbat * 16
                return nc

            nc = lax.fori_loop(0, nch // 2, two_chunks, jnp.int32(0))

            @pl.when(nc > 0)
            def _():
                unpack_fire(0, csrca, cdla, rowsa, semg)
                consume(nc, csrca, cdla, rowsa, semg)

            def srow(row, _):
                zr = row * 0
                dv = denom[pl.ds(row * hs, 16)]
                dinv[pl.ds(0, 16)] = 1.0 / (dv + 1e-16)
                if uniform:
                    for hv in range(heads):
                        s = dinv[pl.ds(zr + hv, 16)][0]
                        for v in range(hv * vph, (hv + 1) * vph):
                            outb[row, pl.ds(v * 16, 16)] = (
                                outb[row, pl.ds(v * 16, 16)] * s)
                else:
                    cph = d_feat // heads
                    nst = (cph + 15) // 16
                    for hv in range(heads):
                        s = dinv[pl.ds(zr + hv, 16)][0]
                        sv = jnp.full((16,), s, jnp.float32)
                        for t in range(nst):
                            wbuf[pl.ds(zr + hv * cph + t * 16, 16)] = sv
                    for v in range(nv):
                        outb[row, pl.ds(v * 16, 16)] = (
                            outb[row, pl.ds(v * 16, 16)]
                            * wbuf[pl.ds(v * 16, 16)])
                return 0
            lax.fori_loop(0, rpr, srow, 0)

            pltpu.sync_copy(outb, out_hbm.at[pl.ds(d0, rpr)])
            return 0

        lax.fori_loop(0, rpt, range_body, 0)

    return k(hext, altail, pk)


def _expand_a(a):
    heads, ch = a.shape
    eye = jnp.eye(heads, dtype=jnp.float32)
    return (a[:, :, None] * eye[:, None, :]).reshape(heads * ch, heads)


def kernel(x, edge_index, W0, a_src0, a_dst0, b0, g0, be0,
           W1, a_src1, a_dst1, b1, g1, be1, W2, a_src2, a_dst2, b2):
    n = x.shape[0]
    loop = jnp.arange(n, dtype=jnp.int32)
    src = jnp.concatenate([edge_index[0].astype(jnp.int32), loop])
    dst = jnp.concatenate([edge_index[1].astype(jnp.int32), loop])
    pk = jnp.bitwise_or(src, dst << 16)
    e_tot = pk.shape[0]
    e_pad = ((e_tot + 2 * _CH - 1) // (2 * _CH)) * (2 * _CH)
    pk = jnp.pad(pk, (0, e_pad - e_tot), constant_values=_N_PAD << 16)

    xp = jnp.pad(x, ((0, _N_PAD - n), (0, 0)))
    row = lambda v: v.reshape(1, -1)

    hext, tail = _mm0(xp, W0, _expand_a(a_src0), _expand_a(a_dst0))
    agg = _sc_layer(hext, tail, pk, heads=4, d_feat=1024, rpr=64,
                    n_ranges=160, e_pad=e_pad)
    hext, tail = _mid(agg, row(b0), row(g0), row(be0), W1,
                      _expand_a(a_src1), _expand_a(a_dst1))
    agg = _sc_layer(hext, tail, pk, heads=4, d_feat=1024, rpr=64,
                    n_ranges=160, e_pad=e_pad)
    hext, tail = _mid(agg, row(b1), row(g1), row(be1), W2,
                      _expand_a(a_src2), _expand_a(a_dst2))
    agg = _sc_layer(hext, tail, pk, heads=6, d_feat=240, rpr=160,
                    n_ranges=64, e_pad=e_pad)
    out = _final(agg, row(b2))
    return out[:n]


# speedup vs baseline: 1.1730x; 1.1730x over previous
import functools

import jax
import jax.numpy as jnp
from jax import lax
from jax.experimental import pallas as pl
from jax.experimental.pallas import tpu as pltpu
from jax.experimental.pallas import tpu_sc as plsc

_N_PAD = 10240
_NEG = 0.2
_CH = 448


def _mm0_body(x_ref, w_ref, as_ref, ad_ref, o_ref, t_ref):
    h = jnp.dot(x_ref[...], w_ref[...], preferred_element_type=jnp.float32)
    als = jnp.dot(h, as_ref[...], preferred_element_type=jnp.float32)
    ald = jnp.dot(h, ad_ref[...], preferred_element_type=jnp.float32)
    bn, m = h.shape
    d_row = o_ref.shape[1]
    z = jnp.zeros((bn, 8 - als.shape[1]), jnp.float32)
    tail = jnp.concatenate([als, z, ald, z], axis=1)
    t_ref[...] = jnp.concatenate([tail, jnp.zeros((bn, 16), jnp.float32)],
                                 axis=1)
    parts = [h, tail]
    if d_row > m + 16:
        parts.append(jnp.zeros((bn, d_row - m - 16), jnp.float32))
    o_ref[...] = jnp.concatenate(parts, axis=1)


def _mm0(x, w, a_s, a_d, bn=512):
    n, k = x.shape
    m = w.shape[1]
    heads = a_s.shape[1]
    d_row = ((m + 16 + 127) // 128) * 128
    return pl.pallas_call(
        _mm0_body,
        grid=(n // bn,),
        in_specs=[pl.BlockSpec((bn, k), lambda i: (i, 0)),
                  pl.BlockSpec((k, m), lambda i: (0, 0)),
                  pl.BlockSpec((m, heads), lambda i: (0, 0)),
                  pl.BlockSpec((m, heads), lambda i: (0, 0))],
        out_specs=[pl.BlockSpec((bn, d_row), lambda i: (i, 0)),
                   pl.BlockSpec((bn, 32), lambda i: (i, 0))],
        out_shape=[jax.ShapeDtypeStruct((n, d_row), jnp.float32),
                   jax.ShapeDtypeStruct((n, 32), jnp.float32)],
    )(x, w, a_s, a_d)


def _mid_body(x_ref, b_ref, g_ref, be_ref, w_ref, as_ref, ad_ref,
              o_ref, t_ref):
    t = x_ref[...] + b_ref[...]
    mu = jnp.mean(t, axis=-1, keepdims=True)
    var = jnp.mean((t - mu) * (t - mu), axis=-1, keepdims=True)
    t = (t - mu) * lax.rsqrt(var + 1e-5) * g_ref[...] + be_ref[...]
    t = jnp.where(t > 0, t, jnp.exp(jnp.minimum(t, 0.0)) - 1.0)
    h = jnp.dot(t, w_ref[...], preferred_element_type=jnp.float32)
    als = jnp.dot(h, as_ref[...], preferred_element_type=jnp.float32)
    ald = jnp.dot(h, ad_ref[...], preferred_element_type=jnp.float32)
    bn, m = h.shape
    d_row = o_ref.shape[1]
    z = jnp.zeros((bn, 8 - als.shape[1]), jnp.float32)
    tail = jnp.concatenate([als, z, ald, z], axis=1)
    t_ref[...] = jnp.concatenate([tail, jnp.zeros((bn, 16), jnp.float32)],
                                 axis=1)
    parts = [h, tail]
    if d_row > m + 16:
        parts.append(jnp.zeros((bn, d_row - m - 16), jnp.float32))
    o_ref[...] = jnp.concatenate(parts, axis=1)


def _mid(x, b, g, be, w, a_s, a_d, bn=512):
    n, k = x.shape
    m = w.shape[1]
    heads = a_s.shape[1]
    d_row = ((m + 16 + 127) // 128) * 128
    vec = pl.BlockSpec((1, k), lambda i: (0, 0))
    return pl.pallas_call(
        _mid_body,
        grid=(n // bn,),
        in_specs=[pl.BlockSpec((bn, k), lambda i: (i, 0)), vec, vec, vec,
                  pl.BlockSpec((k, m), lambda i: (0, 0)),
                  pl.BlockSpec((m, heads), lambda i: (0, 0)),
                  pl.BlockSpec((m, heads), lambda i: (0, 0))],
        out_specs=[pl.BlockSpec((bn, d_row), lambda i: (i, 0)),
                   pl.BlockSpec((bn, 32), lambda i: (i, 0))],
        out_shape=[jax.ShapeDtypeStruct((n, d_row), jnp.float32),
                   jax.ShapeDtypeStruct((n, 32), jnp.float32)],
    )(x, b, g, be, w, a_s, a_d)


def _final_body(x_ref, b_ref, o_ref):
    s = x_ref[:, 0:40]
    for j in range(1, 6):
        s = s + x_ref[:, 40 * j:40 * j + 40]
    t = s * (1.0 / 6.0) + b_ref[...]
    m = jnp.max(t, axis=-1, keepdims=True)
    u = t - m
    o_ref[...] = u - jnp.log(jnp.sum(jnp.exp(u), axis=-1, keepdims=True))


def _final(x, b, bn=1024):
    n = x.shape[0]
    return pl.pallas_call(
        _final_body,
        grid=(n // bn,),
        in_specs=[pl.BlockSpec((bn, 256), lambda i: (i, 0)),
                  pl.BlockSpec((1, 40), lambda i: (0, 0))],
        out_specs=pl.BlockSpec((bn, 40), lambda i: (i, 0)),
        out_shape=jax.ShapeDtypeStruct((n, 40), jnp.float32),
    )(x, b)


def _sc_layer(hext, altail, pk, *, heads, d_feat, rpr, n_ranges, e_pad):
    d_row = hext.shape[1]
    hs = 4 if heads <= 4 else 8
    nv = d_feat // 16
    uniform = (d_feat // heads) % 16 == 0
    vph = (d_feat // heads) // 16 if uniform else 0
    rpt = n_ranges // 32
    nch = e_pad // _CH
    qcap = _CH + 48
    qdump = _CH + 32

    mesh = plsc.VectorSubcoreMesh(core_axis_name="c", subcore_axis_name="s",
                                  num_cores=2, num_subcores=16)

    @functools.partial(
        pl.kernel,
        out_type=jax.ShapeDtypeStruct((_N_PAD, d_feat), jnp.float32),
        mesh=mesh,
        compiler_params=pltpu.CompilerParams(needs_layout_passes=False),
        scratch_types=[
            pltpu.VMEM((_CH,), jnp.int32),
            pltpu.VMEM((_CH,), jnp.int32),
            pltpu.VMEM((qcap,), jnp.int32),
            pltpu.VMEM((16,), jnp.int32),
            pltpu.VMEM((32,), jnp.int32),
            pltpu.VMEM((16, d_row), jnp.float32),
            pltpu.VMEM((rpr, 32), jnp.float32),
            pltpu.VMEM((rpr, d_feat), jnp.float32),
            pltpu.VMEM((rpr * hs + 16,), jnp.float32),
            pltpu.VMEM((32,), jnp.float32),
            pltpu.VMEM((16 if (d_feat // heads) % 16 == 0 else d_feat + 32,), jnp.float32),
            pltpu.SemaphoreType.DMA,
            pltpu.SemaphoreType.DMA,
            pltpu.SemaphoreType.DMA,
        ],
    )
    def k(hext_hbm, altail_hbm, pk_hbm, out_hbm, pka, pkb, cpk, csrca,
          cdla, rowsa, ald2d, outb, denom, pe,
          wbuf, sema, semb, semg):
        wid = lax.axis_index("s") * 2 + lax.axis_index("c")
        iota = lax.iota(jnp.int32, 16)
        zf = jnp.zeros((16,), jnp.float32)
        zi = jnp.zeros((16,), jnp.int32)

        def _zq(i, _):
            cpk[pl.ds(i * 16, 16)] = zi
            return 0
        lax.fori_loop(0, qcap // 16, _zq, 0)

        def range_body(kk, _):
            r = wid * rpt + kk
            d0 = r * rpr

            def _zo(i, _):
                for v in range(nv):
                    outb[i, pl.ds(v * 16, 16)] = zf
                return 0
            lax.fori_loop(0, rpr, _zo, 0)

            def _zd(i, _):
                denom[pl.ds(i * 16, 16)] = zf
                return 0
            lax.fori_loop(0, (rpr * hs + 16) // 16, _zd, 0)

            pltpu.sync_copy(altail_hbm.at[pl.ds(d0, rpr)], ald2d)

            lo = d0 << 16

            def unpack_fire(b, cs, cd, rw, sm):
                cpkv = cpk[pl.ds(b * 16, 16)]
                cs[...] = lax.bitwise_and(cpkv, 0xFFFF)
                cd[pl.ds(0, 16)] = lax.shift_right_logical(cpkv, 16) - d0
                pltpu.async_copy(hext_hbm.at[cs], rw, sm)

            def edge_work(cnt, cd, rw):
                def edge(j, _):
                    iot = lax.iota(jnp.int32, 16)
                    hmk = jnp.where(iot < heads, jnp.int32(1), jnp.int32(0))
                    zj = j * 0
                    dl = cd[pl.ds(j, 16)][0]
                    tail = rw[j, pl.ds(d_feat, 16)]
                    aldv = ald2d[dl, pl.ds(8, 16)]
                    e = tail + aldv
                    e = jnp.maximum(e, 0.0) + _NEG * jnp.minimum(e, 0.0)
                    p = jnp.exp(jnp.minimum(e, 60.0))
                    didx = rpr * hs + iot + hmk * ((dl - rpr) * hs)
                    plsc.addupdate_scatter(denom, [didx], p)
                    pe[pl.ds(0, 16)] = p
                    if uniform:
                        for hv in range(heads):
                            ah = pe[pl.ds(zj + hv, 16)][0]
                            for v in range(hv * vph, (hv + 1) * vph):
                                plsc.addupdate(
                                    outb.at[dl, pl.ds(v * 16, 16)],
                                    rw[j, pl.ds(v * 16, 16)] * ah)
                    else:
                        cph = d_feat // heads
                        nst = (cph + 15) // 16
                        for hv in range(heads):
                            ah = pe[pl.ds(zj + hv, 16)][0]
                            av = jnp.full((16,), ah, jnp.float32)
                            for t in range(nst):
                                wbuf[pl.ds(zj + hv * cph + t * 16, 16)] = av
                        for v in range(nv):
                            plsc.addupdate(
                                outb.at[dl, pl.ds(v * 16, 16)],
                                rw[j, pl.ds(v * 16, 16)]
                                * wbuf[pl.ds(v * 16, 16)])
                    return 0
                lax.fori_loop(0, cnt, edge, 0)

            def consume(cnt, cs, cd, rw, sm):
                pltpu.make_async_copy(hext_hbm.at[cs], rw, sm).wait()
                edge_work(cnt, cd, rw)

            pltpu.async_copy(pk_hbm.at[pl.ds(0, _CH)], pka, sema)

            def two_chunks(i, nc):
                for bsel in range(2):
                    buf = pka if bsel == 0 else pkb
                    sem = sema if bsel == 0 else semb
                    obuf = pkb if bsel == 0 else pka
                    osem = semb if bsel == 0 else sema
                    c = i * 2 + bsel
                    pltpu.make_async_copy(
                        pk_hbm.at[pl.ds(c * _CH, _CH)], buf, sem).wait()

                    @pl.when(c + 1 < nch)
                    def _():
                        pltpu.async_copy(
                            pk_hbm.at[pl.ds((c + 1) * _CH, _CH)], obuf, osem)

                    def compact(g, nc2):
                        iot = lax.iota(jnp.int32, 16)
                        v = buf[pl.ds(g * 16, 16)]
                        u = (v - lo).astype(jnp.uint32)
                        m = u < jnp.uint32(rpr << 16)
                        mi = jnp.where(m, jnp.int32(1), jnp.int32(0))
                        acc = plsc.cumsum(mi)
                        base = qdump + iot
                        idx = base + mi * (nc2 + acc - 1 - base)
                        plsc.store_scatter(cpk, [idx], v)
                        return nc2 + acc[15]
                    nc = lax.fori_loop(0, _CH // 16, compact, nc)

                    nbat = lax.div(nc, jnp.int32(16))

                    def bat(b, _):
                        unpack_fire(b, csrca, cdla, rowsa, semg)
                        consume(16, csrca, cdla, rowsa, semg)
                        return 0
                    lax.fori_loop(0, nbat, bat, 0)

                    lv = cpk[pl.ds(nbat * 16, 16)]
                    cpk[pl.ds(0, 16)] = lv
                    nc = nc - nbat * 16
                return nc

            nc = lax.fori_loop(0, nch // 2, two_chunks, jnp.int32(0))

            @pl.when(nc > 0)
            def _():
                unpack_fire(0, csrca, cdla, rowsa, semg)
                consume(nc, csrca, cdla, rowsa, semg)

            def srow(row, _):
                zr = row * 0
                dv = denom[pl.ds(row * hs, 16)]
                pe[pl.ds(0, 16)] = 1.0 / (dv + 1e-16)
                if uniform:
                    for hv in range(heads):
                        s = pe[pl.ds(zr + hv, 16)][0]
                        for v in range(hv * vph, (hv + 1) * vph):
                            outb[row, pl.ds(v * 16, 16)] = (
                                outb[row, pl.ds(v * 16, 16)] * s)
                else:
                    cph = d_feat // heads
                    nst = (cph + 15) // 16
                    for hv in range(heads):
                        s = pe[pl.ds(zr + hv, 16)][0]
                        sv = jnp.full((16,), s, jnp.float32)
                        for t in range(nst):
                            wbuf[pl.ds(zr + hv * cph + t * 16, 16)] = sv
                    for v in range(nv):
                        outb[row, pl.ds(v * 16, 16)] = (
                            outb[row, pl.ds(v * 16, 16)]
                            * wbuf[pl.ds(v * 16, 16)])
                return 0
            lax.fori_loop(0, rpr, srow, 0)

            pltpu.sync_copy(outb, out_hbm.at[pl.ds(d0, rpr)])
            return 0

        lax.fori_loop(0, rpt, range_body, 0)

    return k(hext, altail, pk)


def _expand_a(a):
    heads, ch = a.shape
    eye = jnp.eye(heads, dtype=jnp.float32)
    return (a[:, :, None] * eye[:, None, :]).reshape(heads * ch, heads)


def kernel(x, edge_index, W0, a_src0, a_dst0, b0, g0, be0,
           W1, a_src1, a_dst1, b1, g1, be1, W2, a_src2, a_dst2, b2):
    n = x.shape[0]
    loop = jnp.arange(n, dtype=jnp.int32)
    src = jnp.concatenate([edge_index[0].astype(jnp.int32), loop])
    dst = jnp.concatenate([edge_index[1].astype(jnp.int32), loop])
    pk = jnp.bitwise_or(src, dst << 16)
    e_tot = pk.shape[0]
    e_pad = ((e_tot + 2 * _CH - 1) // (2 * _CH)) * (2 * _CH)
    pk = jnp.pad(pk, (0, e_pad - e_tot), constant_values=_N_PAD << 16)

    xp = jnp.pad(x, ((0, _N_PAD - n), (0, 0)))
    row = lambda v: v.reshape(1, -1)

    hext, tail = _mm0(xp, W0, _expand_a(a_src0), _expand_a(a_dst0))
    agg = _sc_layer(hext, tail, pk, heads=4, d_feat=1024, rpr=80,
                    n_ranges=128, e_pad=e_pad)
    hext, tail = _mid(agg, row(b0), row(g0), row(be0), W1,
                      _expand_a(a_src1), _expand_a(a_dst1))
    agg = _sc_layer(hext, tail, pk, heads=4, d_feat=1024, rpr=80,
                    n_ranges=128, e_pad=e_pad)
    hext, tail = _mid(agg, row(b1), row(g1), row(be1), W2,
                      _expand_a(a_src2), _expand_a(a_dst2))
    agg = _sc_layer(hext, tail, pk, heads=6, d_feat=240, rpr=160,
                    n_ranges=64, e_pad=e_pad)
    out = _final(agg, row(b2))
    return out[:n]


# speedup vs baseline: 1.2313x; 1.0498x over previous
import functools

import jax
import jax.numpy as jnp
from jax import lax
from jax.experimental import pallas as pl
from jax.experimental.pallas import tpu as pltpu
from jax.experimental.pallas import tpu_sc as plsc

_N_PAD = 10240
_NEG = 0.2
_CH = 1792


def _mm0_body(x_ref, w_ref, as_ref, ad_ref, o_ref, t_ref):
    h = jnp.dot(x_ref[...], w_ref[...], preferred_element_type=jnp.float32)
    als = jnp.dot(h, as_ref[...], preferred_element_type=jnp.float32)
    ald = jnp.dot(h, ad_ref[...], preferred_element_type=jnp.float32)
    bn, m = h.shape
    d_row = o_ref.shape[1]
    z = jnp.zeros((bn, 8 - als.shape[1]), jnp.float32)
    tail = jnp.concatenate([als, z, ald, z], axis=1)
    t_ref[...] = jnp.concatenate([tail, jnp.zeros((bn, 16), jnp.float32)],
                                 axis=1)
    parts = [h, tail]
    if d_row > m + 16:
        parts.append(jnp.zeros((bn, d_row - m - 16), jnp.float32))
    o_ref[...] = jnp.concatenate(parts, axis=1)


def _mm0(x, w, a_s, a_d, bn=512):
    n, k = x.shape
    m = w.shape[1]
    heads = a_s.shape[1]
    d_row = ((m + 16 + 127) // 128) * 128
    return pl.pallas_call(
        _mm0_body,
        grid=(n // bn,),
        in_specs=[pl.BlockSpec((bn, k), lambda i: (i, 0)),
                  pl.BlockSpec((k, m), lambda i: (0, 0)),
                  pl.BlockSpec((m, heads), lambda i: (0, 0)),
                  pl.BlockSpec((m, heads), lambda i: (0, 0))],
        out_specs=[pl.BlockSpec((bn, d_row), lambda i: (i, 0)),
                   pl.BlockSpec((bn, 32), lambda i: (i, 0))],
        out_shape=[jax.ShapeDtypeStruct((n, d_row), jnp.float32),
                   jax.ShapeDtypeStruct((n, 32), jnp.float32)],
    )(x, w, a_s, a_d)


def _mid_body(x_ref, b_ref, g_ref, be_ref, w_ref, as_ref, ad_ref,
              o_ref, t_ref):
    t = x_ref[...] + b_ref[...]
    mu = jnp.mean(t, axis=-1, keepdims=True)
    var = jnp.mean((t - mu) * (t - mu), axis=-1, keepdims=True)
    t = (t - mu) * lax.rsqrt(var + 1e-5) * g_ref[...] + be_ref[...]
    t = jnp.where(t > 0, t, jnp.exp(jnp.minimum(t, 0.0)) - 1.0)
    h = jnp.dot(t, w_ref[...], preferred_element_type=jnp.float32)
    als = jnp.dot(h, as_ref[...], preferred_element_type=jnp.float32)
    ald = jnp.dot(h, ad_ref[...], preferred_element_type=jnp.float32)
    bn, m = h.shape
    d_row = o_ref.shape[1]
    z = jnp.zeros((bn, 8 - als.shape[1]), jnp.float32)
    tail = jnp.concatenate([als, z, ald, z], axis=1)
    t_ref[...] = jnp.concatenate([tail, jnp.zeros((bn, 16), jnp.float32)],
                                 axis=1)
    parts = [h, tail]
    if d_row > m + 16:
        parts.append(jnp.zeros((bn, d_row - m - 16), jnp.float32))
    o_ref[...] = jnp.concatenate(parts, axis=1)


def _mid(x, b, g, be, w, a_s, a_d, bn=512):
    n, k = x.shape
    m = w.shape[1]
    heads = a_s.shape[1]
    d_row = ((m + 16 + 127) // 128) * 128
    vec = pl.BlockSpec((1, k), lambda i: (0, 0))
    return pl.pallas_call(
        _mid_body,
        grid=(n // bn,),
        in_specs=[pl.BlockSpec((bn, k), lambda i: (i, 0)), vec, vec, vec,
                  pl.BlockSpec((k, m), lambda i: (0, 0)),
                  pl.BlockSpec((m, heads), lambda i: (0, 0)),
                  pl.BlockSpec((m, heads), lambda i: (0, 0))],
        out_specs=[pl.BlockSpec((bn, d_row), lambda i: (i, 0)),
                   pl.BlockSpec((bn, 32), lambda i: (i, 0))],
        out_shape=[jax.ShapeDtypeStruct((n, d_row), jnp.float32),
                   jax.ShapeDtypeStruct((n, 32), jnp.float32)],
    )(x, b, g, be, w, a_s, a_d)


def _final_body(x_ref, b_ref, o_ref):
    s = x_ref[:, 0:40]
    for j in range(1, 6):
        s = s + x_ref[:, 40 * j:40 * j + 40]
    t = s * (1.0 / 6.0) + b_ref[...]
    m = jnp.max(t, axis=-1, keepdims=True)
    u = t - m
    o_ref[...] = u - jnp.log(jnp.sum(jnp.exp(u), axis=-1, keepdims=True))


def _final(x, b, bn=1024):
    n = x.shape[0]
    return pl.pallas_call(
        _final_body,
        grid=(n // bn,),
        in_specs=[pl.BlockSpec((bn, 256), lambda i: (i, 0)),
                  pl.BlockSpec((1, 40), lambda i: (0, 0))],
        out_specs=pl.BlockSpec((bn, 40), lambda i: (i, 0)),
        out_shape=jax.ShapeDtypeStruct((n, 40), jnp.float32),
    )(x, b)


def _sc_layer(hext, altail, pk, *, heads, d_feat, rpr, n_ranges, e_pad):
    d_row = hext.shape[1]
    hs = 4 if heads <= 4 else 8
    nv = d_feat // 16
    uniform = (d_feat // heads) % 16 == 0
    vph = (d_feat // heads) // 16 if uniform else 0
    rpt = n_ranges // 32
    nch = e_pad // _CH
    qcap = _CH + 48
    qdump = _CH + 32

    mesh = plsc.VectorSubcoreMesh(core_axis_name="c", subcore_axis_name="s",
                                  num_cores=2, num_subcores=16)

    @functools.partial(
        pl.kernel,
        out_type=jax.ShapeDtypeStruct((_N_PAD, d_feat), jnp.float32),
        mesh=mesh,
        compiler_params=pltpu.CompilerParams(needs_layout_passes=False),
        scratch_types=[
            pltpu.VMEM((_CH,), jnp.int32),
            pltpu.VMEM((_CH,), jnp.int32),
            pltpu.VMEM((qcap,), jnp.int32),
            pltpu.VMEM((16,), jnp.int32),
            pltpu.VMEM((32,), jnp.int32),
            pltpu.VMEM((16, d_row), jnp.float32),
            pltpu.VMEM((rpr, 32), jnp.float32),
            pltpu.VMEM((rpr, d_feat), jnp.float32),
            pltpu.VMEM((rpr * hs + 16,), jnp.float32),
            pltpu.VMEM((32,), jnp.float32),
            pltpu.VMEM((16 if (d_feat // heads) % 16 == 0 else d_feat + 32,), jnp.float32),
            pltpu.SemaphoreType.DMA,
            pltpu.SemaphoreType.DMA,
            pltpu.SemaphoreType.DMA,
        ],
    )
    def k(hext_hbm, altail_hbm, pk_hbm, out_hbm, pka, pkb, cpk, csrca,
          cdla, rowsa, ald2d, outb, denom, pe,
          wbuf, sema, semb, semg):
        wid = lax.axis_index("s") * 2 + lax.axis_index("c")
        iota = lax.iota(jnp.int32, 16)
        zf = jnp.zeros((16,), jnp.float32)
        zi = jnp.zeros((16,), jnp.int32)

        def _zq(i, _):
            cpk[pl.ds(i * 16, 16)] = zi
            return 0
        lax.fori_loop(0, qcap // 16, _zq, 0)

        def range_body(kk, _):
            r = wid * rpt + kk
            d0 = r * rpr

            def _zo(i, _):
                for v in range(nv):
                    outb[i, pl.ds(v * 16, 16)] = zf
                return 0
            lax.fori_loop(0, rpr, _zo, 0)

            def _zd(i, _):
                denom[pl.ds(i * 16, 16)] = zf
                return 0
            lax.fori_loop(0, (rpr * hs + 16) // 16, _zd, 0)

            pltpu.sync_copy(altail_hbm.at[pl.ds(d0, rpr)], ald2d)

            lo = d0 << 16

            def unpack_fire(b, cs, cd, rw, sm):
                cpkv = cpk[pl.ds(b * 16, 16)]
                cs[...] = lax.bitwise_and(cpkv, 0xFFFF)
                cd[pl.ds(0, 16)] = lax.shift_right_logical(cpkv, 16) - d0
                pltpu.async_copy(hext_hbm.at[cs], rw, sm)

            def edge_work(cnt, cd, rw):
                def edge(j, _):
                    iot = lax.iota(jnp.int32, 16)
                    hmk = jnp.where(iot < heads, jnp.int32(1), jnp.int32(0))
                    zj = j * 0
                    dl = cd[pl.ds(j, 16)][0]
                    tail = rw[j, pl.ds(d_feat, 16)]
                    aldv = ald2d[dl, pl.ds(8, 16)]
                    e = tail + aldv
                    e = jnp.maximum(e, 0.0) + _NEG * jnp.minimum(e, 0.0)
                    p = jnp.exp(jnp.minimum(e, 60.0))
                    didx = rpr * hs + iot + hmk * ((dl - rpr) * hs)
                    plsc.addupdate_scatter(denom, [didx], p)
                    pe[pl.ds(0, 16)] = p
                    if uniform:
                        for hv in range(heads):
                            ah = pe[pl.ds(zj + hv, 16)][0]
                            for v in range(hv * vph, (hv + 1) * vph):
                                plsc.addupdate(
                                    outb.at[dl, pl.ds(v * 16, 16)],
                                    rw[j, pl.ds(v * 16, 16)] * ah)
                    else:
                        cph = d_feat // heads
                        nst = (cph + 15) // 16
                        for hv in range(heads):
                            ah = pe[pl.ds(zj + hv, 16)][0]
                            av = jnp.full((16,), ah, jnp.float32)
                            for t in range(nst):
                                wbuf[pl.ds(zj + hv * cph + t * 16, 16)] = av
                        for v in range(nv):
                            plsc.addupdate(
                                outb.at[dl, pl.ds(v * 16, 16)],
                                rw[j, pl.ds(v * 16, 16)]
                                * wbuf[pl.ds(v * 16, 16)])
                    return 0
                lax.fori_loop(0, cnt, edge, 0)

            def consume(cnt, cs, cd, rw, sm):
                pltpu.make_async_copy(hext_hbm.at[cs], rw, sm).wait()
                edge_work(cnt, cd, rw)

            pltpu.async_copy(pk_hbm.at[pl.ds(0, _CH)], pka, sema)

            def two_chunks(i, nc):
                for bsel in range(2):
                    buf = pka if bsel == 0 else pkb
                    sem = sema if bsel == 0 else semb
                    obuf = pkb if bsel == 0 else pka
                    osem = semb if bsel == 0 else sema
                    c = i * 2 + bsel
                    pltpu.make_async_copy(
                        pk_hbm.at[pl.ds(c * _CH, _CH)], buf, sem).wait()

                    @pl.when(c + 1 < nch)
                    def _():
                        pltpu.async_copy(
                            pk_hbm.at[pl.ds((c + 1) * _CH, _CH)], obuf, osem)

                    def compact(g, nc2):
                        iot = lax.iota(jnp.int32, 16)
                        v = buf[pl.ds(g * 16, 16)]
                        u = (v - lo).astype(jnp.uint32)
                        m = u < jnp.uint32(rpr << 16)
                        mi = jnp.where(m, jnp.int32(1), jnp.int32(0))
                        acc = plsc.cumsum(mi)
                        base = qdump + iot
                        idx = base + mi * (nc2 + acc - 1 - base)
                        plsc.store_scatter(cpk, [idx], v)
                        return nc2 + acc[15]
                    nc = lax.fori_loop(0, _CH // 16, compact, nc)

                    nbat = lax.div(nc, jnp.int32(16))

                    def bat(b, _):
                        unpack_fire(b, csrca, cdla, rowsa, semg)
                        consume(16, csrca, cdla, rowsa, semg)
                        return 0
                    lax.fori_loop(0, nbat, bat, 0)

                    lv = cpk[pl.ds(nbat * 16, 16)]
                    cpk[pl.ds(0, 16)] = lv
                    nc = nc - nbat * 16
                return nc

            nc = lax.fori_loop(0, nch // 2, two_chunks, jnp.int32(0))

            @pl.when(nc > 0)
            def _():
                unpack_fire(0, csrca, cdla, rowsa, semg)
                consume(nc, csrca, cdla, rowsa, semg)

            def srow(row, _):
                zr = row * 0
                dv = denom[pl.ds(row * hs, 16)]
                pe[pl.ds(0, 16)] = 1.0 / (dv + 1e-16)
                if uniform:
                    for hv in range(heads):
                        s = pe[pl.ds(zr + hv, 16)][0]
                        for v in range(hv * vph, (hv + 1) * vph):
                            outb[row, pl.ds(v * 16, 16)] = (
                                outb[row, pl.ds(v * 16, 16)] * s)
                else:
                    cph = d_feat // heads
                    nst = (cph + 15) // 16
                    for hv in range(heads):
                        s = pe[pl.ds(zr + hv, 16)][0]
                        sv = jnp.full((16,), s, jnp.float32)
                        for t in range(nst):
                            wbuf[pl.ds(zr + hv * cph + t * 16, 16)] = sv
                    for v in range(nv):
                        outb[row, pl.ds(v * 16, 16)] = (
                            outb[row, pl.ds(v * 16, 16)]
                            * wbuf[pl.ds(v * 16, 16)])
                return 0
            lax.fori_loop(0, rpr, srow, 0)

            pltpu.sync_copy(outb, out_hbm.at[pl.ds(d0, rpr)])
            return 0

        lax.fori_loop(0, rpt, range_body, 0)

    return k(hext, altail, pk)


def _expand_a(a):
    heads, ch = a.shape
    eye = jnp.eye(heads, dtype=jnp.float32)
    return (a[:, :, None] * eye[:, None, :]).reshape(heads * ch, heads)


def kernel(x, edge_index, W0, a_src0, a_dst0, b0, g0, be0,
           W1, a_src1, a_dst1, b1, g1, be1, W2, a_src2, a_dst2, b2):
    n = x.shape[0]
    loop = jnp.arange(n, dtype=jnp.int32)
    src = jnp.concatenate([edge_index[0].astype(jnp.int32), loop])
    dst = jnp.concatenate([edge_index[1].astype(jnp.int32), loop])
    pk = jnp.bitwise_or(src, dst << 16)
    e_tot = pk.shape[0]
    e_pad = ((e_tot + 2 * _CH - 1) // (2 * _CH)) * (2 * _CH)
    pk = jnp.pad(pk, (0, e_pad - e_tot), constant_values=_N_PAD << 16)

    xp = jnp.pad(x, ((0, _N_PAD - n), (0, 0)))
    row = lambda v: v.reshape(1, -1)

    hext, tail = _mm0(xp, W0, _expand_a(a_src0), _expand_a(a_dst0))
    agg = _sc_layer(hext, tail, pk, heads=4, d_feat=1024, rpr=80,
                    n_ranges=128, e_pad=e_pad)
    hext, tail = _mid(agg, row(b0), row(g0), row(be0), W1,
                      _expand_a(a_src1), _expand_a(a_dst1))
    agg = _sc_layer(hext, tail, pk, heads=4, d_feat=1024, rpr=80,
                    n_ranges=128, e_pad=e_pad)
    hext, tail = _mid(agg, row(b1), row(g1), row(be1), W2,
                      _expand_a(a_src2), _expand_a(a_dst2))
    agg = _sc_layer(hext, tail, pk, heads=6, d_feat=240, rpr=160,
                    n_ranges=64, e_pad=e_pad)
    out = _final(agg, row(b2))
    return out[:n]
